# baseline (speedup 1.0000x reference)
import functools

import jax
import jax.numpy as jnp
from jax import lax
from jax.experimental import pallas as pl
from jax.experimental.pallas import tpu as pltpu


def _chamfer_tile_kernel(x1_ref, x2ts_ref, d1_ref, i1_ref, d2_ref, i2_ref,
                         *, tn, n, m):
    ni = pl.program_id(1)
    x1 = x1_ref[0]
    x2ts = x2ts_ref[0]

    inner2 = jnp.dot(x1, x2ts, preferred_element_type=jnp.float32)
    xn = jnp.sum(x1 * x1, axis=1, keepdims=True)
    yn = jnp.sum(x2ts * x2ts, axis=0, keepdims=True) * 0.25
    d = jnp.maximum((xn + yn) + inner2, 0.0)

    def _tree_min(parts):
        while len(parts) > 1:
            nxt = [jnp.minimum(parts[k], parts[k + 1])
                   for k in range(0, len(parts) - 1, 2)]
            if len(parts) % 2:
                nxt.append(parts[-1])
            parts = nxt
        return parts[0]

    nc = m // 128
    cs = [d[:, j * 128:(j + 1) * 128] for j in range(nc)]
    run = _tree_min(cs)
    idxc = jnp.zeros((tn, 128), jnp.int32)
    for j in range(nc - 1, -1, -1):
        idxc = jnp.where(cs[j] == run, j, idxc)
    rmin = jnp.min(run, axis=1, keepdims=True)
    lane = lax.broadcasted_iota(jnp.int32, (tn, 128), 1)
    cand = idxc * 128 + lane
    imin = jnp.min(jnp.where(run == rmin, cand, m), axis=1,
                   keepdims=True)
    d1_ref[0] = rmin
    i1_ref[0] = imin

    nr = tn // 8
    rows = [d[i * 8:(i + 1) * 8, :] for i in range(nr)]
    run2 = _tree_min(rows)
    idxr = jnp.zeros((8, m), jnp.int32)
    for i in range(nr - 1, -1, -1):
        idxr = jnp.where(rows[i] == run2, i, idxr)
    cmin = jnp.min(run2, axis=0, keepdims=True)
    sub = lax.broadcasted_iota(jnp.int32, (8, m), 0)
    cand2 = idxr * 8 + sub
    cidx = jnp.min(jnp.where(run2 == cmin, cand2, n), axis=0,
                   keepdims=True) + ni * tn

    @pl.when(ni == 0)
    def _init():
        d2_ref[0] = cmin
        i2_ref[0] = cidx

    @pl.when(ni != 0)
    def _merge():
        prev_d = d2_ref[0]
        prev_i = i2_ref[0]
        better = cmin < prev_d
        d2_ref[0] = jnp.where(better, cmin, prev_d)
        i2_ref[0] = jnp.where(better, cidx, prev_i)


def _chamfer(xyz1, xyz2, tn):
    b, n, _ = xyz1.shape
    m = xyz2.shape[1]
    f32 = jnp.float32
    i32 = jnp.int32

    pad = jnp.zeros((b, n, 5), f32)
    x1p = jnp.concatenate([xyz1, pad], axis=-1)
    x2ts = jnp.concatenate([jnp.swapaxes(xyz2, 1, 2) * -2.0,
                            jnp.zeros((b, 5, m), f32)], axis=1)

    grid = (b, n // tn)
    d1, i1, d2, i2 = pl.pallas_call(
        functools.partial(_chamfer_tile_kernel, tn=tn, n=n, m=m),
        grid=grid,
        in_specs=[
            pl.BlockSpec((1, tn, 8), lambda bi, ni: (bi, ni, 0)),
            pl.BlockSpec((1, 8, m), lambda bi, ni: (bi, 0, 0)),
        ],
        out_specs=[
            pl.BlockSpec((1, tn, 1), lambda bi, ni: (bi, ni, 0)),
            pl.BlockSpec((1, tn, 1), lambda bi, ni: (bi, ni, 0)),
            pl.BlockSpec((1, 1, m), lambda bi, ni: (bi, 0, 0)),
            pl.BlockSpec((1, 1, m), lambda bi, ni: (bi, 0, 0)),
        ],
        out_shape=[
            jax.ShapeDtypeStruct((b, n, 1), f32),
            jax.ShapeDtypeStruct((b, n, 1), i32),
            jax.ShapeDtypeStruct((b, 1, m), f32),
            jax.ShapeDtypeStruct((b, 1, m), i32),
        ],
        compiler_params=pltpu.CompilerParams(
            dimension_semantics=("parallel", "arbitrary"),
        ),
    )(x1p, x2ts)

    return (d1.reshape(b, n), d2.reshape(b, m),
            i1.reshape(b, n), i2.reshape(b, m))


def kernel(xyz1, xyz2):
    return _chamfer(xyz1, xyz2, tn=256)


# speedup vs baseline: 1.2803x; 1.2803x over previous
import functools

import jax
import jax.numpy as jnp
from jax import lax
from jax.experimental import pallas as pl
from jax.experimental.pallas import tpu as pltpu


def _tree_min(parts):
    while len(parts) > 1:
        nxt = [jnp.minimum(parts[k], parts[k + 1])
               for k in range(0, len(parts) - 1, 2)]
        if len(parts) % 2:
            nxt.append(parts[-1])
        parts = nxt
    return parts[0]


def _chamfer_tile_kernel(x1_ref, x2ts_ref, d1_ref, i1_ref, d2_ref, i2_ref,
                         *, tn, n, m, nblk):
    ni = pl.program_id(1)
    x1 = x1_ref[0]
    x2ts = x2ts_ref[0]

    xn = jnp.sum(x1 * x1, axis=1, keepdims=True)
    yn = jnp.sum(x2ts * x2ts, axis=0, keepdims=True) * 0.25

    bw = m // nblk
    nr = tn // 8
    sub = lax.broadcasted_iota(jnp.int32, (8, bw), 0)
    cs = []
    runp = []
    run2s, rowss = [], []

    for bi_ in range(nblk):
        sl = slice(bi_ * bw, (bi_ + 1) * bw)
        innerb = jnp.dot(x1, x2ts[:, sl],
                         preferred_element_type=jnp.float32)
        db = (xn + yn[:, sl]) + innerb
        bcs = [db[:, j * 128:(j + 1) * 128] for j in range(bw // 128)]
        cs += bcs
        runp.append(_tree_min(bcs))
        rows = [db[i * 8:(i + 1) * 8, :] for i in range(nr)]
        rowss.append(rows)
        run2s.append(jnp.maximum(_tree_min(rows), 0.0))

    cmins, cidxs = [], []
    for bi_ in range(nblk):
        run2 = run2s[bi_]
        rows = rowss[bi_]
        idxr = jnp.zeros((8, bw), jnp.int32)
        for i in range(nr - 1, -1, -1):
            idxr = jnp.where(rows[i] <= run2, i, idxr)
        cmins.append(jnp.min(run2, axis=0, keepdims=True))
        cand2 = idxr * 8 + sub
        cidxs.append(jnp.min(jnp.where(run2 == cmins[-1], cand2, n), axis=0,
                             keepdims=True) + ni * tn)

    cmin = jnp.concatenate(cmins, axis=1)
    cidx = jnp.concatenate(cidxs, axis=1)

    @pl.when(ni == 0)
    def _init():
        d2_ref[0] = cmin
        i2_ref[0] = cidx

    @pl.when(ni != 0)
    def _merge():
        prev_d = d2_ref[0]
        prev_i = i2_ref[0]
        better = cmin < prev_d
        d2_ref[0] = jnp.where(better, cmin, prev_d)
        i2_ref[0] = jnp.where(better, cidx, prev_i)

    nc = m // 128
    run = jnp.maximum(_tree_min(runp), 0.0)
    idxc = jnp.zeros((tn, 128), jnp.int32)
    for j in range(nc - 1, -1, -1):
        idxc = jnp.where(cs[j] <= run, j, idxc)
    rmin = jnp.min(run, axis=1, keepdims=True)
    lane = lax.broadcasted_iota(jnp.int32, (tn, 128), 1)
    cand = idxc * 128 + lane
    imin = jnp.min(jnp.where(run == rmin, cand, m), axis=1,
                   keepdims=True)
    d1_ref[0] = rmin
    i1_ref[0] = imin


def _chamfer(xyz1, xyz2, tn, nblk):
    b, n, _ = xyz1.shape
    m = xyz2.shape[1]
    f32 = jnp.float32
    i32 = jnp.int32

    pad = jnp.zeros((b, n, 5), f32)
    x1p = jnp.concatenate([xyz1, pad], axis=-1)
    x2ts = jnp.concatenate([jnp.swapaxes(xyz2, 1, 2) * -2.0,
                            jnp.zeros((b, 5, m), f32)], axis=1)

    tn = min(tn, n)
    nblk = max(1, min(nblk, m // 128))
    grid = (b, n // tn)
    d1, i1, d2, i2 = pl.pallas_call(
        functools.partial(_chamfer_tile_kernel, tn=tn, n=n, m=m, nblk=nblk),
        grid=grid,
        in_specs=[
            pl.BlockSpec((1, tn, 8), lambda bi, ni: (bi, ni, 0)),
            pl.BlockSpec((1, 8, m), lambda bi, ni: (bi, 0, 0)),
        ],
        out_specs=[
            pl.BlockSpec((1, tn, 1), lambda bi, ni: (bi, ni, 0)),
            pl.BlockSpec((1, tn, 1), lambda bi, ni: (bi, ni, 0)),
            pl.BlockSpec((1, 1, m), lambda bi, ni: (bi, 0, 0)),
            pl.BlockSpec((1, 1, m), lambda bi, ni: (bi, 0, 0)),
        ],
        out_shape=[
            jax.ShapeDtypeStruct((b, n, 1), f32),
            jax.ShapeDtypeStruct((b, n, 1), i32),
            jax.ShapeDtypeStruct((b, 1, m), f32),
            jax.ShapeDtypeStruct((b, 1, m), i32),
        ],
        compiler_params=pltpu.CompilerParams(
            dimension_semantics=("parallel", "arbitrary"),
        ),
    )(x1p, x2ts)

    return (d1.reshape(b, n), d2.reshape(b, m),
            i1.reshape(b, n), i2.reshape(b, m))


def kernel(xyz1, xyz2):
    return _chamfer(xyz1, xyz2, tn=2048, nblk=8)
